# baseline (speedup 1.0000x reference)
import functools

import jax
import jax.numpy as jnp
from jax import lax
from jax.experimental import pallas as pl
from jax.experimental.pallas import tpu as pltpu
from jax.experimental.pallas import tpu_sc as plsc

V = 1000
NB = 1024
NT = 50
NC, NS = 2, 16
NW = NC * NS
NBAT = NB // NW
CPI = 64
CG = 56


def _lse_body(tab_ref, lse_ref):
    x = tab_ref[...]
    m = jnp.max(x, axis=1)
    s = jnp.sum(jnp.exp(x - m[:, None]), axis=1)
    lse_ref[...] = m + jnp.log(s)


def _compute_lse(table):
    return pl.pallas_call(
        _lse_body,
        out_shape=jax.ShapeDtypeStruct((V,), jnp.float32),
    )(table)


_sc_mesh = plsc.VectorSubcoreMesh(core_axis_name="c", subcore_axis_name="s")


@functools.partial(
    pl.kernel,
    out_type=[
        jax.ShapeDtypeStruct((NB, NT, V), jnp.float32),
        jax.ShapeDtypeStruct((NW, 16), jnp.float32),
    ],
    mesh=_sc_mesh,
    compiler_params=pltpu.CompilerParams(
        use_tc_tiling_on_sc=False, needs_layout_passes=False),
    scratch_types=[
        pltpu.VMEM((NBAT, CPI), jnp.int32),
        pltpu.VMEM((NBAT, CPI), jnp.int32),
        [pltpu.VMEM((CG, V), jnp.float32)] * 2,
        pltpu.VMEM((V,), jnp.float32),
        pltpu.VMEM((16,), jnp.float32),
        [pltpu.SemaphoreType.DMA] * 2,
        [pltpu.SemaphoreType.DMA] * 2,
    ],
)
def _sc_gather(idx_hbm, tgt_hbm, lse_hbm, table_hbm, out_hbm, part_hbm,
               idx_v, tgt_v, rows_v, lse_v, acc_v, gsem, ssem):
    cid = lax.axis_index("c")
    sid = lax.axis_index("s")
    wid = sid * NC + cid
    bbase = wid * NBAT

    pltpu.sync_copy(lse_hbm, lse_v)
    pltpu.sync_copy(idx_hbm.at[wid], idx_v)
    pltpu.sync_copy(tgt_hbm.at[wid], tgt_v)

    def start_gather(c, s):
        pltpu.async_copy(
            table_hbm.at[idx_v.at[c, pl.ds(0, CG)]], rows_v[s], gsem[s])

    def wait_gather(c, s):
        pltpu.make_async_copy(
            table_hbm.at[idx_v.at[c, pl.ds(0, CG)]], rows_v[s], gsem[s]).wait()

    def start_scatter(c, s):
        pltpu.async_copy(
            rows_v[s].at[pl.ds(0, NT)], out_hbm.at[bbase + c], ssem[s])

    def wait_scatter(c, s):
        pltpu.make_async_copy(
            rows_v[s].at[pl.ds(0, NT)], out_hbm.at[bbase + c], ssem[s]).wait()

    iota16 = lax.iota(jnp.int32, 16)

    def compute(acc, c, s):
        for j in range(3):
            rid = iota16 + j * 16
            tg = tgt_v[c, pl.ds(j * 16, 16)]
            ii = idx_v[c, pl.ds(j * 16, 16)]
            tv = plsc.load_gather(rows_v[s], [rid, tg])
            lv = plsc.load_gather(lse_v, [ii])
            acc = acc + (lv - tv)
        rid = jnp.minimum(iota16 + 48, CG - 1)
        tg = tgt_v[c, pl.ds(48, 16)]
        ii = idx_v[c, pl.ds(48, 16)]
        tv = plsc.load_gather(rows_v[s], [rid, tg])
        lv = plsc.load_gather(lse_v, [ii])
        return acc + jnp.where(iota16 < NT - 48, lv - tv, 0.0)

    start_gather(0, 0)
    start_gather(1, 1)

    @pl.loop(0, NBAT - 2, step=2, init_carry=jnp.zeros((16,), jnp.float32))
    def acc_loop(g, acc):
        for s in range(2):
            c = g + s
            wait_gather(c, s)
            acc = compute(acc, c, s)
            start_scatter(c, s)
            wait_scatter(c, s)
            start_gather(c + 2, s)
        return acc

    acc = acc_loop
    for s in range(2):
        c = NBAT - 2 + s
        wait_gather(c, s)
        acc = compute(acc, c, s)
        pltpu.sync_copy(rows_v[s].at[pl.ds(0, NT)], out_hbm.at[bbase + c])

    acc_v[...] = acc
    pltpu.sync_copy(acc_v, part_hbm.at[wid])


def kernel(idx, targets, table):
    idx3 = idx.reshape(NW, NBAT, NT).astype(jnp.int32)
    tgt3 = targets.reshape(NW, NBAT, NT).astype(jnp.int32)
    pad = [(0, 0), (0, 0), (0, CPI - NT)]
    idx_p = jnp.pad(idx3, pad)
    tgt_p = jnp.pad(tgt3, pad)
    lse = _compute_lse(table)
    logits, partials = _sc_gather(idx_p, tgt_p, lse, table)
    loss = jnp.sum(partials) / jnp.float32(NB * NT)
    return logits, loss


# speedup vs baseline: 1.3784x; 1.3784x over previous
import functools

import jax
import jax.numpy as jnp
from jax import lax
from jax.experimental import pallas as pl
from jax.experimental.pallas import tpu as pltpu
from jax.experimental.pallas import tpu_sc as plsc

V = 1000
NB = 1024
NT = 50
NC, NS = 2, 16
NW = NC * NS
NBAT = NB // NW
CPI = 64
CG = 56
VP = 1024


def _lse_body(tab_ref, lse_ref):
    x = tab_ref[...]
    m = jnp.max(x, axis=1)
    s = jnp.sum(jnp.exp(x - m[:, None]), axis=1)
    lse_ref[...] = m + jnp.log(s)


def _compute_lse(table):
    return pl.pallas_call(
        _lse_body,
        out_shape=jax.ShapeDtypeStruct((V,), jnp.float32),
    )(table)


_sc_mesh = plsc.VectorSubcoreMesh(core_axis_name="c", subcore_axis_name="s")


@functools.partial(
    pl.kernel,
    out_type=[
        jax.ShapeDtypeStruct((NB, CG, VP), jnp.float32),
        jax.ShapeDtypeStruct((NW, 16), jnp.float32),
    ],
    mesh=_sc_mesh,
    compiler_params=pltpu.CompilerParams(use_tc_tiling_on_sc=True),
    scratch_types=[
        pltpu.VMEM((NBAT * CPI,), jnp.int32),
        pltpu.VMEM((NBAT * CPI,), jnp.int32),
        [pltpu.VMEM((CG, VP), jnp.float32)] * 2,
        [pltpu.VMEM((CG,), jnp.float32)] * 2,
        [pltpu.VMEM((CG,), jnp.float32)] * 2,
        pltpu.VMEM((16,), jnp.float32),
        [pltpu.SemaphoreType.DMA] * 2,
        [pltpu.SemaphoreType.DMA] * 2,
        [pltpu.SemaphoreType.DMA] * 2,
        [pltpu.SemaphoreType.DMA] * 2,
    ],
)
def _sc_gather(idx_hbm, fli_hbm, lse_hbm, table_hbm, tabf_hbm,
               out_hbm, part_hbm,
               idx_v, fli_v, rows_v, tv_v, lv_v, acc_v,
               gsem, ssem, tsem, lsem):
    cid = lax.axis_index("c")
    sid = lax.axis_index("s")
    wid = sid * NC + cid
    bbase = wid * NBAT

    pltpu.sync_copy(idx_hbm.at[pl.ds(wid * NBAT * CPI, NBAT * CPI)], idx_v)
    pltpu.sync_copy(fli_hbm.at[pl.ds(wid * NBAT * CPI, NBAT * CPI)], fli_v)

    def start_gathers(c, s):
        ii = idx_v.at[pl.ds(c * CPI, CG)]
        fl = fli_v.at[pl.ds(c * CPI, CG)]
        pltpu.async_copy(table_hbm.at[ii], rows_v[s], gsem[s])
        pltpu.async_copy(tabf_hbm.at[fl], tv_v[s], tsem[s])
        pltpu.async_copy(lse_hbm.at[ii], lv_v[s], lsem[s])

    def wait_gathers(c, s):
        ii = idx_v.at[pl.ds(c * CPI, CG)]
        fl = fli_v.at[pl.ds(c * CPI, CG)]
        pltpu.make_async_copy(table_hbm.at[ii], rows_v[s], gsem[s]).wait()
        pltpu.make_async_copy(tabf_hbm.at[fl], tv_v[s], tsem[s]).wait()
        pltpu.make_async_copy(lse_hbm.at[ii], lv_v[s], lsem[s]).wait()

    def start_scatter(c, s):
        pltpu.async_copy(rows_v[s], out_hbm.at[bbase + c], ssem[s])

    def wait_scatter(c, s):
        pltpu.make_async_copy(rows_v[s], out_hbm.at[bbase + c], ssem[s]).wait()

    iota16 = lax.iota(jnp.int32, 16)

    def compute(acc, s):
        for j in range(3):
            tv = tv_v[s][pl.ds(j * 16, 16)]
            lv = lv_v[s][pl.ds(j * 16, 16)]
            acc = acc + (lv - tv)
        tv = tv_v[s][pl.ds(40, 16)]
        lv = lv_v[s][pl.ds(40, 16)]
        keep = jnp.logical_and(iota16 >= 8, iota16 < 10)
        return acc + jnp.where(keep, lv - tv, 0.0)

    start_gathers(0, 0)
    start_gathers(1, 1)

    @pl.loop(0, NBAT - 2, step=2, init_carry=jnp.zeros((16,), jnp.float32))
    def acc_loop(g, acc):
        for s in range(2):
            c = g + s
            wait_gathers(c, s)
            acc = compute(acc, s)
            start_scatter(c, s)
            wait_scatter(c, s)
            start_gathers(c + 2, s)
        return acc

    acc = acc_loop
    for s in range(2):
        c = NBAT - 2 + s
        wait_gathers(c, s)
        acc = compute(acc, s)
        pltpu.sync_copy(rows_v[s], out_hbm.at[bbase + c])

    acc_v[...] = acc
    pltpu.sync_copy(acc_v, part_hbm.at[wid])


def kernel(idx, targets, table):
    idx3 = idx.reshape(NW, NBAT, NT).astype(jnp.int32)
    tgt3 = targets.reshape(NW, NBAT, NT).astype(jnp.int32)
    fli3 = idx3 * V + tgt3
    pad = [(0, 0), (0, 0), (0, CPI - NT)]
    idx_f = jnp.pad(idx3, pad).reshape(-1)
    fli_f = jnp.pad(fli3, pad).reshape(-1)
    lse = _compute_lse(table)
    table_p = jnp.pad(table, ((0, 0), (0, VP - V)))
    logits_p, partials = _sc_gather(
        idx_f, fli_f, lse, table_p, table.reshape(-1))
    loss = jnp.sum(partials) / jnp.float32(NB * NT)
    return logits_p[:, :NT, :V], loss
